# baseline (speedup 1.0000x reference)
import functools

import jax
import jax.numpy as jnp
from jax import lax
from jax.experimental import pallas as pl
from jax.experimental.pallas import tpu as pltpu
from jax.experimental.pallas import tpu_sc as plsc

_H = 24
_L = _H * _H
_MASK_RATIO = 0.6
_LEN_KEEP = int(_L * (1.0 - _MASK_RATIO))
_ROWS = 8


def _mask_body(n_ref, out_ref):
    n = n_ref[...]
    b = lax.bitcast_convert_type(n, jnp.int32)
    Bn = b.shape[0]
    k0 = jnp.full((Bn, 1), _LEN_KEEP, jnp.int32)

    def descent(bits, nbits, valid, k_init):
        def step(i, carry):
            prefix, k = carry
            bit = nbits - 1 - i
            hi = lax.shift_right_logical(bits, bit + 1)
            phi = lax.shift_right_logical(prefix, bit + 1)
            b0 = lax.shift_right_logical(bits, bit) & 1
            sel = valid & (hi == phi) & (b0 == 0)
            c = jnp.sum(sel.astype(jnp.int32), axis=1, keepdims=True)
            take1 = k > c
            k = jnp.where(take1, k - c, k)
            prefix = jnp.where(take1, prefix | (1 << bit), prefix)
            return prefix, k
        prefix, _ = lax.fori_loop(0, nbits, step,
                                  (jnp.zeros((Bn, 1), jnp.int32), k_init))
        return prefix

    t = descent(b, 30, jnp.full(b.shape, True), k0)
    lt = b < t
    cnt_lt = jnp.sum(lt.astype(jnp.int32), axis=1, keepdims=True)
    eq = b == t
    need = k0 - cnt_lt
    idx = lax.broadcasted_iota(jnp.int32, b.shape, 1)
    it = descent(idx, 10, eq, need)
    keep = lt | (eq & (idx <= it))
    out_ref[...] = keep.astype(jnp.float32)


_NV = _L // 16


def _sc_mask_body(noise_hbm, out_hbm, nv, ov, buf):
    info = plsc.get_sparse_core_info()
    wid = lax.axis_index("s") * info.num_cores + lax.axis_index("c")
    rows = 2
    base = wid * rows
    pltpu.sync_copy(noise_hbm.at[pl.ds(base, rows)], nv)

    lanes = lax.broadcasted_iota(jnp.int32, (16,), 0)
    one = jnp.full((16,), 1, jnp.int32)
    zero = jnp.full((16,), 0, jnp.int32)

    def splat_total(acc):
        buf[pl.ds(16, 16)] = zero
        cur = acc
        for s in (1, 2, 4, 8):
            buf[pl.ds(0, 16)] = cur
            cur = cur + buf[pl.ds(s, 16)]
        cur = jnp.where(lanes == 0, cur, zero)
        buf[pl.ds(0, 16)] = zero
        for s in (1, 2, 4, 8):
            buf[pl.ds(16, 16)] = cur
            cur = cur + buf[pl.ds(16 - s, 16)]
        return cur

    for r in range(rows):
        def bits_of(j):
            return lax.bitcast_convert_type(nv[r, pl.ds(j * 16, 16)],
                                            jnp.int32)

        def count_le(get_key, get_valid, mid):
            acc = zero
            for j in range(_NV):
                le = jnp.where(get_key(j) <= mid, one, zero)
                acc = acc + get_valid(j) * le
            return splat_total(acc)

        def kth_smallest(get_key, get_valid, k, hi0, iters):
            def step(_, carry):
                lo, hi = carry
                mid = jnp.right_shift(lo + hi, 1)
                c = count_le(get_key, get_valid, mid)
                ge = c >= k
                hi = jnp.where(ge, mid, hi)
                lo = jnp.where(ge, lo, mid + 1)
                return lo, hi
            lo, _ = lax.fori_loop(0, iters, step, (zero, hi0))
            return lo

        def valid_all(j):
            return one

        k0 = jnp.full((16,), _LEN_KEEP, jnp.int32)
        t = kth_smallest(bits_of, valid_all, k0,
                         jnp.full((16,), 1 << 30, jnp.int32), 30)

        cnt_lt = count_le(bits_of, valid_all, t - 1)
        need = k0 - cnt_lt

        def idx_of(j):
            return lanes + j * 16

        def valid_eq(j):
            return jnp.where(bits_of(j) == t, one, zero)

        it = kth_smallest(idx_of, valid_eq, need,
                          jnp.full((16,), _L, jnp.int32), 10)

        fone = jnp.full((16,), 1.0, jnp.float32)
        fzero = jnp.full((16,), 0.0, jnp.float32)
        for j in range(_NV):
            b = bits_of(j)
            ltf = jnp.where(b < t, fone, fzero)
            eqf = jnp.where(b == t, fone, fzero)
            lef = jnp.where(idx_of(j) <= it, fone, fzero)
            ov[r, pl.ds(j * 16, 16)] = ltf + eqf * lef

    pltpu.sync_copy(ov, out_hbm.at[pl.ds(base, rows)])


def _sc_mask(noise):
    B = noise.shape[0]
    mesh = plsc.VectorSubcoreMesh(core_axis_name="c", subcore_axis_name="s")
    return pl.kernel(
        _sc_mask_body,
        mesh=mesh,
        out_type=jax.ShapeDtypeStruct((B, _L), jnp.float32),
        scratch_types=[
            pltpu.VMEM((2, _L), jnp.float32),
            pltpu.VMEM((2, _L), jnp.float32),
            pltpu.VMEM((32,), jnp.int32),
        ],
    )(noise)


def _expand(k, m):
    s = _H * k
    a0 = lax.broadcasted_iota(jnp.int32, (s, _H), 0)
    a1 = lax.broadcasted_iota(jnp.int32, (s, _H), 1)
    A = (a0 // k == a1).astype(jnp.float32)
    b0 = lax.broadcasted_iota(jnp.int32, (_H, s), 0)
    b1 = lax.broadcasted_iota(jnp.int32, (_H, s), 1)
    Bt = (b0 == b1 // k).astype(jnp.float32)
    t = jnp.dot(A, m, preferred_element_type=jnp.float32)
    return jnp.dot(t, Bt, preferred_element_type=jnp.float32)


_AB = 4


def _apply_body(m_ref, x_ref, y_ref, o24_ref, o48_ref, o96_ref,
                o192_ref, o384_ref):
    for b in range(_AB):
        m24 = m_ref[b]
        m48 = _expand(2, m24)
        m96 = _expand(4, m24)
        m192 = _expand(8, m24)
        m384 = _expand(16, m24)
        o24_ref[b, 0] = m24 > 0.5
        o48_ref[b, 0] = m48 > 0.5
        o96_ref[b, 0] = m96 > 0.5
        o192_ref[b, 0] = m192 > 0.5
        o384_ref[b, 0] = m384 > 0.5
        y_ref[b] = x_ref[b] * m384[None]


def kernel(inp_bchw):
    B, C, Hh, Ww = inp_bchw.shape
    noise = jax.random.uniform(jax.random.key(42), (B, _L), dtype=jnp.float32)

    mask_flat = _sc_mask(noise)

    m2d = mask_flat.reshape(B, _H, _H)

    out_shapes = (
        jax.ShapeDtypeStruct((B, C, Hh, Ww), jnp.float32),
        jax.ShapeDtypeStruct((B, 1, _H, _H), jnp.bool_),
        jax.ShapeDtypeStruct((B, 1, 2 * _H, 2 * _H), jnp.bool_),
        jax.ShapeDtypeStruct((B, 1, 4 * _H, 4 * _H), jnp.bool_),
        jax.ShapeDtypeStruct((B, 1, 8 * _H, 8 * _H), jnp.bool_),
        jax.ShapeDtypeStruct((B, 1, 16 * _H, 16 * _H), jnp.bool_),
    )
    lvl_spec = lambda s: pl.BlockSpec((_AB, 1, s, s), lambda b: (b, 0, 0, 0))
    masked, l24, l48, l96, l192, l384 = pl.pallas_call(
        _apply_body,
        grid=(B // _AB,),
        in_specs=[
            pl.BlockSpec((_AB, _H, _H), lambda b: (b, 0, 0)),
            pl.BlockSpec((_AB, C, Hh, Ww), lambda b: (b, 0, 0, 0)),
        ],
        out_specs=[
            pl.BlockSpec((_AB, C, Hh, Ww), lambda b: (b, 0, 0, 0)),
            lvl_spec(_H), lvl_spec(2 * _H), lvl_spec(4 * _H),
            lvl_spec(8 * _H), lvl_spec(16 * _H),
        ],
        out_shape=out_shapes,
        compiler_params=pltpu.CompilerParams(
            dimension_semantics=("parallel",)),
    )(m2d, inp_bchw)

    return (masked, l24, l48, l96, l192, l384)


# speedup vs baseline: 1.0221x; 1.0221x over previous
import functools

import jax
import jax.numpy as jnp
from jax import lax
from jax.experimental import pallas as pl
from jax.experimental.pallas import tpu as pltpu
from jax.experimental.pallas import tpu_sc as plsc

_H = 24
_L = _H * _H
_MASK_RATIO = 0.6
_LEN_KEEP = int(_L * (1.0 - _MASK_RATIO))
_ROWS = 8


def _mask_body(n_ref, out_ref):
    n = n_ref[...]
    b = lax.bitcast_convert_type(n, jnp.int32)
    Bn = b.shape[0]
    k0 = jnp.full((Bn, 1), _LEN_KEEP, jnp.int32)

    def descent(bits, nbits, valid, k_init):
        def step(i, carry):
            prefix, k = carry
            bit = nbits - 1 - i
            hi = lax.shift_right_logical(bits, bit + 1)
            phi = lax.shift_right_logical(prefix, bit + 1)
            b0 = lax.shift_right_logical(bits, bit) & 1
            sel = valid & (hi == phi) & (b0 == 0)
            c = jnp.sum(sel.astype(jnp.int32), axis=1, keepdims=True)
            take1 = k > c
            k = jnp.where(take1, k - c, k)
            prefix = jnp.where(take1, prefix | (1 << bit), prefix)
            return prefix, k
        prefix, _ = lax.fori_loop(0, nbits, step,
                                  (jnp.zeros((Bn, 1), jnp.int32), k_init))
        return prefix

    t = descent(b, 30, jnp.full(b.shape, True), k0)
    lt = b < t
    cnt_lt = jnp.sum(lt.astype(jnp.int32), axis=1, keepdims=True)
    eq = b == t
    need = k0 - cnt_lt
    idx = lax.broadcasted_iota(jnp.int32, b.shape, 1)
    it = descent(idx, 10, eq, need)
    keep = lt | (eq & (idx <= it))
    out_ref[...] = keep.astype(jnp.float32)


_NV = _L // 16


def _sc_mask_body(noise_hbm, out_hbm, nv, ov, buf):
    info = plsc.get_sparse_core_info()
    wid = lax.axis_index("s") * info.num_cores + lax.axis_index("c")
    rows = 2
    base = wid * rows
    pltpu.sync_copy(noise_hbm.at[pl.ds(base, rows)], nv)

    lanes = lax.broadcasted_iota(jnp.int32, (16,), 0)
    one = jnp.full((16,), 1, jnp.int32)
    zero = jnp.full((16,), 0, jnp.int32)

    def splat_total2(a0, a1):
        buf[pl.ds(16, 16)] = zero
        buf[pl.ds(48, 16)] = zero
        c0, c1 = a0, a1
        for s in (1, 2, 4, 8):
            buf[pl.ds(0, 16)] = c0
            buf[pl.ds(32, 16)] = c1
            c0 = c0 + buf[pl.ds(s, 16)]
            c1 = c1 + buf[pl.ds(32 + s, 16)]
        c0 = jnp.where(lanes == 0, c0, zero)
        c1 = jnp.where(lanes == 0, c1, zero)
        buf[pl.ds(0, 16)] = zero
        buf[pl.ds(32, 16)] = zero
        for s in (1, 2, 4, 8):
            buf[pl.ds(16, 16)] = c0
            buf[pl.ds(48, 16)] = c1
            c0 = c0 + buf[pl.ds(16 - s, 16)]
            c1 = c1 + buf[pl.ds(48 - s, 16)]
        return c0, c1

    def bits_of(r, j):
        return lax.bitcast_convert_type(nv[r, pl.ds(j * 16, 16)],
                                        jnp.int32)

    def count_le2(get_key, get_valid, m0, m1):
        a0 = zero
        a1 = zero
        for j in range(_NV):
            a0 = a0 + get_valid(0, j) * jnp.where(get_key(0, j) <= m0,
                                                  one, zero)
            a1 = a1 + get_valid(1, j) * jnp.where(get_key(1, j) <= m1,
                                                  one, zero)
        return splat_total2(a0, a1)

    def kth_smallest2(get_key, get_valid, ka, kb, hi_init, iters):
        def step(_, carry):
            lo0, hi0, lo1, hi1 = carry
            m0 = jnp.right_shift(lo0 + hi0, 1)
            m1 = jnp.right_shift(lo1 + hi1, 1)
            c0, c1 = count_le2(get_key, get_valid, m0, m1)
            ge0 = c0 >= ka
            ge1 = c1 >= kb
            return (jnp.where(ge0, lo0, m0 + 1), jnp.where(ge0, m0, hi0),
                    jnp.where(ge1, lo1, m1 + 1), jnp.where(ge1, m1, hi1))
        lo0, _, lo1, _ = lax.fori_loop(0, iters, step,
                                       (zero, hi_init, zero, hi_init))
        return lo0, lo1

    def valid_all(r, j):
        return one

    kk = jnp.full((16,), _LEN_KEEP, jnp.int32)
    t0, t1 = kth_smallest2(bits_of, valid_all, kk, kk,
                           jnp.full((16,), 1 << 30, jnp.int32), 30)

    c0, c1 = count_le2(bits_of, valid_all, t0 - 1, t1 - 1)
    need0, need1 = kk - c0, kk - c1

    def idx_of(r, j):
        return lanes + j * 16

    ts = (t0, t1)

    def valid_eq(r, j):
        return jnp.where(bits_of(r, j) == ts[r], one, zero)

    it0, it1 = kth_smallest2(idx_of, valid_eq, need0, need1,
                             jnp.full((16,), _L, jnp.int32), 10)

    its = (it0, it1)
    fone = jnp.full((16,), 1.0, jnp.float32)
    fzero = jnp.full((16,), 0.0, jnp.float32)
    for j in range(_NV):
        for r in range(rows):
            b = bits_of(r, j)
            ltf = jnp.where(b < ts[r], fone, fzero)
            eqf = jnp.where(b == ts[r], fone, fzero)
            lef = jnp.where(idx_of(r, j) <= its[r], fone, fzero)
            ov[r, pl.ds(j * 16, 16)] = ltf + eqf * lef

    pltpu.sync_copy(ov, out_hbm.at[pl.ds(base, rows)])


def _sc_mask(noise):
    B = noise.shape[0]
    mesh = plsc.VectorSubcoreMesh(core_axis_name="c", subcore_axis_name="s")
    return pl.kernel(
        _sc_mask_body,
        mesh=mesh,
        out_type=jax.ShapeDtypeStruct((B, _L), jnp.float32),
        scratch_types=[
            pltpu.VMEM((2, _L), jnp.float32),
            pltpu.VMEM((2, _L), jnp.float32),
            pltpu.VMEM((64,), jnp.int32),
        ],
    )(noise)


def _expand(k, m):
    s = _H * k
    a0 = lax.broadcasted_iota(jnp.int32, (s, _H), 0)
    a1 = lax.broadcasted_iota(jnp.int32, (s, _H), 1)
    A = (a0 // k == a1).astype(jnp.float32)
    b0 = lax.broadcasted_iota(jnp.int32, (_H, s), 0)
    b1 = lax.broadcasted_iota(jnp.int32, (_H, s), 1)
    Bt = (b0 == b1 // k).astype(jnp.float32)
    t = jnp.dot(A, m, preferred_element_type=jnp.float32)
    return jnp.dot(t, Bt, preferred_element_type=jnp.float32)


_AB = 4


def _apply_body(m_ref, x_ref, y_ref, o24_ref, o48_ref, o96_ref,
                o192_ref, o384_ref):
    for b in range(_AB):
        m24 = m_ref[b]
        m48 = _expand(2, m24)
        m96 = _expand(4, m24)
        m192 = _expand(8, m24)
        m384 = _expand(16, m24)
        o24_ref[b, 0] = m24 > 0.5
        o48_ref[b, 0] = m48 > 0.5
        o96_ref[b, 0] = m96 > 0.5
        o192_ref[b, 0] = m192 > 0.5
        o384_ref[b, 0] = m384 > 0.5
        y_ref[b] = x_ref[b] * m384[None]


def kernel(inp_bchw):
    B, C, Hh, Ww = inp_bchw.shape
    noise = jax.random.uniform(jax.random.key(42), (B, _L), dtype=jnp.float32)

    mask_flat = _sc_mask(noise)

    m2d = mask_flat.reshape(B, _H, _H)

    out_shapes = (
        jax.ShapeDtypeStruct((B, C, Hh, Ww), jnp.float32),
        jax.ShapeDtypeStruct((B, 1, _H, _H), jnp.bool_),
        jax.ShapeDtypeStruct((B, 1, 2 * _H, 2 * _H), jnp.bool_),
        jax.ShapeDtypeStruct((B, 1, 4 * _H, 4 * _H), jnp.bool_),
        jax.ShapeDtypeStruct((B, 1, 8 * _H, 8 * _H), jnp.bool_),
        jax.ShapeDtypeStruct((B, 1, 16 * _H, 16 * _H), jnp.bool_),
    )
    lvl_spec = lambda s: pl.BlockSpec((_AB, 1, s, s), lambda b: (b, 0, 0, 0))
    masked, l24, l48, l96, l192, l384 = pl.pallas_call(
        _apply_body,
        grid=(B // _AB,),
        in_specs=[
            pl.BlockSpec((_AB, _H, _H), lambda b: (b, 0, 0)),
            pl.BlockSpec((_AB, C, Hh, Ww), lambda b: (b, 0, 0, 0)),
        ],
        out_specs=[
            pl.BlockSpec((_AB, C, Hh, Ww), lambda b: (b, 0, 0, 0)),
            lvl_spec(_H), lvl_spec(2 * _H), lvl_spec(4 * _H),
            lvl_spec(8 * _H), lvl_spec(16 * _H),
        ],
        out_shape=out_shapes,
        compiler_params=pltpu.CompilerParams(
            dimension_semantics=("parallel",)),
    )(m2d, inp_bchw)

    return (masked, l24, l48, l96, l192, l384)


# speedup vs baseline: 1.2059x; 1.1799x over previous
import functools

import jax
import jax.numpy as jnp
from jax import lax
from jax.experimental import pallas as pl
from jax.experimental.pallas import tpu as pltpu
from jax.experimental.pallas import tpu_sc as plsc

_H = 24
_L = _H * _H
_MASK_RATIO = 0.6
_LEN_KEEP = int(_L * (1.0 - _MASK_RATIO))
_ROWS = 8


def _mask_body(n_ref, out_ref):
    n = n_ref[...]
    b = lax.bitcast_convert_type(n, jnp.int32)
    Bn = b.shape[0]
    k0 = jnp.full((Bn, 1), _LEN_KEEP, jnp.int32)

    def descent(bits, nbits, valid, k_init):
        def step(i, carry):
            prefix, k = carry
            bit = nbits - 1 - i
            hi = lax.shift_right_logical(bits, bit + 1)
            phi = lax.shift_right_logical(prefix, bit + 1)
            b0 = lax.shift_right_logical(bits, bit) & 1
            sel = valid & (hi == phi) & (b0 == 0)
            c = jnp.sum(sel.astype(jnp.int32), axis=1, keepdims=True)
            take1 = k > c
            k = jnp.where(take1, k - c, k)
            prefix = jnp.where(take1, prefix | (1 << bit), prefix)
            return prefix, k
        prefix, _ = lax.fori_loop(0, nbits, step,
                                  (jnp.zeros((Bn, 1), jnp.int32), k_init))
        return prefix

    t = descent(b, 30, jnp.full(b.shape, True), k0)
    lt = b < t
    cnt_lt = jnp.sum(lt.astype(jnp.int32), axis=1, keepdims=True)
    eq = b == t
    need = k0 - cnt_lt
    idx = lax.broadcasted_iota(jnp.int32, b.shape, 1)
    it = descent(idx, 10, eq, need)
    keep = lt | (eq & (idx <= it))
    out_ref[...] = keep.astype(jnp.float32)


_NV = _L // 16


def _sc_mask_body(noise_hbm, out_hbm, nv, ov, buf):
    info = plsc.get_sparse_core_info()
    wid = lax.axis_index("s") * info.num_cores + lax.axis_index("c")
    rows = 2
    base = wid * rows
    pltpu.sync_copy(noise_hbm.at[pl.ds(base, rows)], nv)

    lanes = lax.broadcasted_iota(jnp.int32, (16,), 0)
    one = jnp.full((16,), 1, jnp.int32)
    zero = jnp.full((16,), 0, jnp.int32)

    def splat_total2(a0, a1):
        buf[pl.ds(16, 16)] = zero
        buf[pl.ds(48, 16)] = zero
        c0, c1 = a0, a1
        for s in (1, 2, 4, 8):
            buf[pl.ds(0, 16)] = c0
            buf[pl.ds(32, 16)] = c1
            c0 = c0 + buf[pl.ds(s, 16)]
            c1 = c1 + buf[pl.ds(32 + s, 16)]
        c0 = jnp.where(lanes == 0, c0, zero)
        c1 = jnp.where(lanes == 0, c1, zero)
        buf[pl.ds(0, 16)] = zero
        buf[pl.ds(32, 16)] = zero
        for s in (1, 2, 4, 8):
            buf[pl.ds(16, 16)] = c0
            buf[pl.ds(48, 16)] = c1
            c0 = c0 + buf[pl.ds(16 - s, 16)]
            c1 = c1 + buf[pl.ds(48 - s, 16)]
        return c0, c1

    def bits_of(r, j):
        return lax.bitcast_convert_type(nv[r, pl.ds(j * 16, 16)],
                                        jnp.int32)

    def count_le2(get_key, get_valid, m0, m1):
        a0 = zero
        a1 = zero
        for j in range(_NV):
            a0 = a0 + get_valid(0, j) * jnp.where(get_key(0, j) <= m0,
                                                  one, zero)
            a1 = a1 + get_valid(1, j) * jnp.where(get_key(1, j) <= m1,
                                                  one, zero)
        return splat_total2(a0, a1)

    def kth_smallest2(get_key, get_valid, ka, kb, hi_init, iters):
        def step(_, carry):
            lo0, hi0, lo1, hi1 = carry
            m0 = jnp.right_shift(lo0 + hi0, 1)
            m1 = jnp.right_shift(lo1 + hi1, 1)
            c0, c1 = count_le2(get_key, get_valid, m0, m1)
            ge0 = c0 >= ka
            ge1 = c1 >= kb
            return (jnp.where(ge0, lo0, m0 + 1), jnp.where(ge0, m0, hi0),
                    jnp.where(ge1, lo1, m1 + 1), jnp.where(ge1, m1, hi1))
        lo0, _, lo1, _ = lax.fori_loop(0, iters, step,
                                       (zero, hi_init, zero, hi_init))
        return lo0, lo1

    def valid_all(r, j):
        return one

    kk = jnp.full((16,), _LEN_KEEP, jnp.int32)
    t0, t1 = kth_smallest2(bits_of, valid_all, kk, kk,
                           jnp.full((16,), 1 << 30, jnp.int32), 30)

    c0, c1 = count_le2(bits_of, valid_all, t0 - 1, t1 - 1)
    need0, need1 = kk - c0, kk - c1

    def idx_of(r, j):
        return lanes + j * 16

    ts = (t0, t1)

    def valid_eq(r, j):
        return jnp.where(bits_of(r, j) == ts[r], one, zero)

    it0, it1 = kth_smallest2(idx_of, valid_eq, need0, need1,
                             jnp.full((16,), _L, jnp.int32), 10)

    its = (it0, it1)
    fone = jnp.full((16,), 1.0, jnp.float32)
    fzero = jnp.full((16,), 0.0, jnp.float32)
    for j in range(_NV):
        for r in range(rows):
            b = bits_of(r, j)
            ltf = jnp.where(b < ts[r], fone, fzero)
            eqf = jnp.where(b == ts[r], fone, fzero)
            lef = jnp.where(idx_of(r, j) <= its[r], fone, fzero)
            ov[r, pl.ds(j * 16, 16)] = ltf + eqf * lef

    pltpu.sync_copy(ov, out_hbm.at[pl.ds(base, rows)])


def _sc_mask(noise):
    B = noise.shape[0]
    mesh = plsc.VectorSubcoreMesh(core_axis_name="c", subcore_axis_name="s")
    return pl.kernel(
        _sc_mask_body,
        mesh=mesh,
        out_type=jax.ShapeDtypeStruct((B, _L), jnp.float32),
        scratch_types=[
            pltpu.VMEM((2, _L), jnp.float32),
            pltpu.VMEM((2, _L), jnp.float32),
            pltpu.VMEM((64,), jnp.int32),
        ],
    )(noise)


def _expand(k, m):
    s = _H * k
    a0 = lax.broadcasted_iota(jnp.int32, (s, _H), 0)
    a1 = lax.broadcasted_iota(jnp.int32, (s, _H), 1)
    A = (a0 // k == a1).astype(jnp.float32)
    b0 = lax.broadcasted_iota(jnp.int32, (_H, s), 0)
    b1 = lax.broadcasted_iota(jnp.int32, (_H, s), 1)
    Bt = (b0 == b1 // k).astype(jnp.float32)
    t = jnp.dot(A, m, preferred_element_type=jnp.float32)
    return jnp.dot(t, Bt, preferred_element_type=jnp.float32)


_AB = 4


def _apply_body(m_ref, x_ref, y_ref, o24_ref, o48_ref, o96_ref,
                o192_ref, o384_ref):
    for b in range(_AB):
        m24 = m_ref[b]
        m48 = _expand(2, m24)
        m96 = _expand(4, m24)
        m192 = _expand(8, m24)
        m384 = _expand(16, m24)
        o24_ref[b, 0] = m24.astype(jnp.int8)
        o48_ref[b, 0] = m48.astype(jnp.int8)
        o96_ref[b, 0] = m96.astype(jnp.int8)
        o192_ref[b, 0] = m192.astype(jnp.int8)
        o384_ref[b, 0] = m384.astype(jnp.int8)
        y_ref[b] = x_ref[b] * m384[None]


def kernel(inp_bchw):
    B, C, Hh, Ww = inp_bchw.shape
    noise = jax.random.uniform(jax.random.key(42), (B, _L), dtype=jnp.float32)

    mask_flat = _sc_mask(noise)

    m2d = mask_flat.reshape(B, _H, _H)

    out_shapes = (
        jax.ShapeDtypeStruct((B, C, Hh, Ww), jnp.float32),
        jax.ShapeDtypeStruct((B, 1, _H, _H), jnp.int8),
        jax.ShapeDtypeStruct((B, 1, 2 * _H, 2 * _H), jnp.int8),
        jax.ShapeDtypeStruct((B, 1, 4 * _H, 4 * _H), jnp.int8),
        jax.ShapeDtypeStruct((B, 1, 8 * _H, 8 * _H), jnp.int8),
        jax.ShapeDtypeStruct((B, 1, 16 * _H, 16 * _H), jnp.int8),
    )
    lvl_spec = lambda s: pl.BlockSpec((_AB, 1, s, s), lambda b: (b, 0, 0, 0))
    masked, l24, l48, l96, l192, l384 = pl.pallas_call(
        _apply_body,
        grid=(B // _AB,),
        in_specs=[
            pl.BlockSpec((_AB, _H, _H), lambda b: (b, 0, 0)),
            pl.BlockSpec((_AB, C, Hh, Ww), lambda b: (b, 0, 0, 0)),
        ],
        out_specs=[
            pl.BlockSpec((_AB, C, Hh, Ww), lambda b: (b, 0, 0, 0)),
            lvl_spec(_H), lvl_spec(2 * _H), lvl_spec(4 * _H),
            lvl_spec(8 * _H), lvl_spec(16 * _H),
        ],
        out_shape=out_shapes,
        compiler_params=pltpu.CompilerParams(
            dimension_semantics=("parallel",)),
    )(m2d, inp_bchw)

    return (masked,
            l24.astype(jnp.bool_), l48.astype(jnp.bool_),
            l96.astype(jnp.bool_), l192.astype(jnp.bool_),
            l384.astype(jnp.bool_))
